# Initial kernel scaffold; baseline (speedup 1.0000x reference)
import jax
import jax.numpy as jnp
from jax.experimental import pallas as pl


def kernel(x, edge_index, W1, W_mu):
    raise NotImplementedError("write your pallas kernel here")


# speedup vs baseline: 44.0767x; 44.0767x over previous
import functools

import jax
import jax.numpy as jnp
from jax import lax
from jax.experimental import pallas as pl
from jax.experimental.pallas import tpu as pltpu
from jax.experimental.pallas import tpu_sc as plsc

N = 10000
E = 320000
DF = 128
DH = 32
DO = 16

NC = 2
NS = 16
NW = NC * NS
CHUNK = 128
NCH = 80
EPW = NCH * CHUNK
EP = NW * EPW
R = 10240
RPT = R // NS

_SC_MESH = plsc.VectorSubcoreMesh(core_axis_name="c", subcore_axis_name="s")


def _deg_body(dst_hbm, out_hbm, dst_v, hist, sem):
    c = lax.axis_index("c")
    s = lax.axis_index("s")
    wid = s * NC + c

    zeros16 = jnp.zeros((16,), jnp.float32)

    def zero_body(i, carry):
        hist[pl.ds(i * 16, 16)] = zeros16
        return carry

    lax.fori_loop(0, R // 16, zero_body, 0)

    pltpu.sync_copy(dst_hbm.at[wid], dst_v)

    ones16 = jnp.ones((16,), jnp.float32)

    def chunk_body(j, carry):
        def vec_body(k, carry2):
            idx = dst_v[j, pl.ds(k * 16, 16)]
            plsc.addupdate_scatter(hist, [idx], ones16)
            return carry2

        return lax.fori_loop(0, CHUNK // 16, vec_body, carry)

    lax.fori_loop(0, NCH, chunk_body, 0)

    pltpu.sync_copy(hist, out_hbm.at[wid])


_deg_call = pl.kernel(
    _deg_body,
    out_type=jax.ShapeDtypeStruct((NW, R), jnp.float32),
    mesh=_SC_MESH,
    compiler_params=pltpu.CompilerParams(needs_layout_passes=False),
    scratch_types=[
        pltpu.VMEM((NCH, CHUNK), jnp.int32),
        pltpu.VMEM((R,), jnp.float32),
        pltpu.SemaphoreType.DMA,
    ],
)


def _make_agg(width):
    def body(hs_hbm, src_hbm, dst_hbm, out_hbm, src_v, dst_v, gbuf, zbuf, acc, sem):
        c = lax.axis_index("c")
        s = lax.axis_index("s")
        wid = s * NC + c

        pltpu.sync_copy(src_hbm.at[wid], src_v)
        pltpu.sync_copy(dst_hbm.at[wid], dst_v)

        zeros16 = jnp.zeros((16,), jnp.float32)

        def zrow(i, carry):
            def zcol(k, carry2):
                zbuf[i, pl.ds(k * 16, 16)] = zeros16
                return carry2

            return lax.fori_loop(0, width // 16, zcol, carry)

        lax.fori_loop(0, RPT, zrow, 0)
        pltpu.sync_copy(zbuf, acc.at[pl.ds(s * RPT, RPT)])
        plsc.subcore_barrier()

        pltpu.make_async_copy(
            hs_hbm.at[src_v.at[0]], gbuf.at[0], sem.at[0]
        ).start()

        def chunk_body(j, carry):
            b = lax.rem(j, 2)
            nxt = j + 1

            @pl.when(nxt < NCH)
            def _start():
                pltpu.make_async_copy(
                    hs_hbm.at[src_v.at[nxt]], gbuf.at[1 - b], sem.at[1 - b]
                ).start()

            pltpu.make_async_copy(
                hs_hbm.at[src_v.at[j]], gbuf.at[b], sem.at[b]
            ).wait()
            pltpu.sync_copy(gbuf.at[b], acc.at[dst_v.at[j]], add=True)
            return carry

        lax.fori_loop(0, NCH, chunk_body, 0)
        plsc.subcore_barrier()

        pltpu.sync_copy(
            acc.at[pl.ds(s * RPT, RPT)], out_hbm.at[c, pl.ds(s * RPT, RPT)]
        )

    return pl.kernel(
        body,
        out_type=jax.ShapeDtypeStruct((NC, R, width), jnp.float32),
        mesh=_SC_MESH,
        compiler_params=pltpu.CompilerParams(use_tc_tiling_on_sc=False),
        scratch_types=[
            pltpu.VMEM((NCH, CHUNK), jnp.int32),
            pltpu.VMEM((NCH, CHUNK), jnp.int32),
            pltpu.VMEM((2, CHUNK, width), jnp.float32),
            pltpu.VMEM((RPT, width), jnp.float32),
            pltpu.VMEM_SHARED((R, width), jnp.float32),
            pltpu.SemaphoreType.DMA((2,)),
        ],
    )


_agg_h = _make_agg(DH)
_agg_o = _make_agg(DO)


_BR = 512
_GRID = R // _BR


def _tc1_body(x_ref, w1_ref, degp_ref, hs_ref):
    deg = jnp.sum(degp_ref[...], axis=0) + 1.0
    dinv = lax.rsqrt(deg)
    h = jnp.dot(x_ref[...], w1_ref[...], preferred_element_type=jnp.float32)
    hs_ref[...] = h * dinv[:, None]


_tc1_call = pl.pallas_call(
    _tc1_body,
    grid=(_GRID,),
    in_specs=[
        pl.BlockSpec((_BR, DF), lambda i: (i, 0)),
        pl.BlockSpec((DF, DH), lambda i: (0, 0)),
        pl.BlockSpec((NW, _BR), lambda i: (0, i)),
    ],
    out_specs=pl.BlockSpec((_BR, DH), lambda i: (i, 0)),
    out_shape=jax.ShapeDtypeStruct((R, DH), jnp.float32),
)


def _tc2_body(acc_ref, hs_ref, degp_ref, wmu_ref, out_ref):
    deg = jnp.sum(degp_ref[...], axis=0) + 1.0
    dinv = lax.rsqrt(deg)
    z = jnp.maximum(
        (acc_ref[0] + acc_ref[1] + hs_ref[...]) * dinv[:, None], 0.0
    )
    h2 = jnp.dot(z, wmu_ref[...], preferred_element_type=jnp.float32)
    out_ref[...] = h2 * dinv[:, None]


_tc2_call = pl.pallas_call(
    _tc2_body,
    grid=(_GRID,),
    in_specs=[
        pl.BlockSpec((NC, _BR, DH), lambda i: (0, i, 0)),
        pl.BlockSpec((_BR, DH), lambda i: (i, 0)),
        pl.BlockSpec((NW, _BR), lambda i: (0, i)),
        pl.BlockSpec((DH, DO), lambda i: (0, 0)),
    ],
    out_specs=pl.BlockSpec((_BR, DO), lambda i: (i, 0)),
    out_shape=jax.ShapeDtypeStruct((R, DO), jnp.float32),
)


def _tc3_body(acc_ref, hs_ref, degp_ref, out_ref):
    deg = jnp.sum(degp_ref[...], axis=0) + 1.0
    dinv = lax.rsqrt(deg)
    out_ref[...] = (acc_ref[0] + acc_ref[1] + hs_ref[...]) * dinv[:, None]


_tc3_call = pl.pallas_call(
    _tc3_body,
    grid=(_GRID,),
    in_specs=[
        pl.BlockSpec((NC, _BR, DO), lambda i: (0, i, 0)),
        pl.BlockSpec((_BR, DO), lambda i: (i, 0)),
        pl.BlockSpec((NW, _BR), lambda i: (0, i)),
    ],
    out_specs=pl.BlockSpec((_BR, DO), lambda i: (i, 0)),
    out_shape=jax.ShapeDtypeStruct((R, DO), jnp.float32),
)


def kernel(x, edge_index, W1, W_mu):
    src = edge_index[0].astype(jnp.int32)
    dst = edge_index[1].astype(jnp.int32)

    npad = EP - E
    pad_idx = N + (jnp.arange(npad, dtype=jnp.int32) % (R - N))
    src_p = jnp.concatenate([src, pad_idx]).reshape(NW, NCH, CHUNK)
    dst_p = jnp.concatenate([dst, pad_idx]).reshape(NW, NCH, CHUNK)
    x_p = jnp.pad(x, ((0, R - N), (0, 0)))

    degp = _deg_call(dst_p)

    hs1 = _tc1_call(x_p, W1, degp)
    acc1 = _agg_h(hs1, src_p, dst_p)

    hs2 = _tc2_call(acc1, hs1, degp, W_mu)
    acc2 = _agg_o(hs2, src_p, dst_p)

    return _tc3_call(acc2, hs2, degp)[:N]


# speedup vs baseline: 50.9817x; 1.1567x over previous
import functools

import jax
import jax.numpy as jnp
from jax import lax
from jax.experimental import pallas as pl
from jax.experimental.pallas import tpu as pltpu
from jax.experimental.pallas import tpu_sc as plsc

N = 10000
E = 320000
DF = 128
DH = 32
DO = 16

NC = 2
NS = 16
NW = NC * NS
CHUNK = 128
NCH = 80
EPW = NCH * CHUNK
EP = NW * EPW
R = 10240
RPT = R // NS

_SC_MESH = plsc.VectorSubcoreMesh(core_axis_name="c", subcore_axis_name="s")


def _deg_body(dst_hbm, out_hbm, dst_v, hist, sem):
    c = lax.axis_index("c")
    s = lax.axis_index("s")
    wid = s * NC + c

    zeros16 = jnp.zeros((16,), jnp.float32)

    def zero_body(i, carry):
        hist[pl.ds(i * 16, 16)] = zeros16
        return carry

    lax.fori_loop(0, R // 16, zero_body, 0)

    pltpu.sync_copy(dst_hbm.at[wid], dst_v)

    ones16 = jnp.ones((16,), jnp.float32)

    def chunk_body(j, carry):
        def vec_body(k, carry2):
            idx = dst_v[j, pl.ds(k * 16, 16)]
            plsc.addupdate_scatter(hist, [idx], ones16)
            return carry2

        return lax.fori_loop(0, CHUNK // 16, vec_body, carry)

    lax.fori_loop(0, NCH, chunk_body, 0)

    pltpu.sync_copy(hist, out_hbm.at[wid])


_deg_call = pl.kernel(
    _deg_body,
    out_type=jax.ShapeDtypeStruct((NW, R), jnp.float32),
    mesh=_SC_MESH,
    compiler_params=pltpu.CompilerParams(needs_layout_passes=False),
    scratch_types=[
        pltpu.VMEM((NCH, CHUNK), jnp.int32),
        pltpu.VMEM((R,), jnp.float32),
        pltpu.SemaphoreType.DMA,
    ],
)


def _make_agg(width):
    def body(hs_hbm, src_hbm, dst_hbm, out_hbm, src_v, dst_v, gbuf, zbuf, acc,
             gsem, ssem):
        c = lax.axis_index("c")
        s = lax.axis_index("s")
        wid = s * NC + c

        pltpu.sync_copy(src_hbm.at[wid], src_v)
        pltpu.sync_copy(dst_hbm.at[wid], dst_v)

        zeros16 = jnp.zeros((16,), jnp.float32)

        def zrow(i, carry):
            def zcol(k, carry2):
                zbuf[i, pl.ds(k * 16, 16)] = zeros16
                return carry2

            return lax.fori_loop(0, width // 16, zcol, carry)

        lax.fori_loop(0, RPT, zrow, 0)
        pltpu.sync_copy(zbuf, acc.at[pl.ds(s * RPT, RPT)])
        plsc.subcore_barrier()

        for p in range(3):
            pltpu.async_copy(hs_hbm.at[src_v.at[p]], gbuf.at[p], gsem.at[p])

        def chunk_body(j, carry):
            b = lax.rem(j, 4)
            pltpu.make_async_copy(
                hs_hbm.at[src_v.at[j]], gbuf.at[b], gsem.at[b]
            ).wait()
            pltpu.async_copy(
                gbuf.at[b], acc.at[dst_v.at[j]], ssem.at[b], add=True
            )
            nxt = j + 3

            @pl.when(nxt < NCH)
            def _start():
                nb = lax.rem(nxt, 4)

                @pl.when(j >= 1)
                def _drain():
                    pltpu.make_async_copy(
                        gbuf.at[nb], acc.at[dst_v.at[j - 1]], ssem.at[nb]
                    ).wait()

                pltpu.async_copy(
                    hs_hbm.at[src_v.at[nxt]], gbuf.at[nb], gsem.at[nb]
                )

            return carry

        lax.fori_loop(0, NCH, chunk_body, 0)

        def drain_body(k, carry):
            j = NCH - 4 + k
            b = lax.rem(j, 4)
            pltpu.make_async_copy(
                gbuf.at[b], acc.at[dst_v.at[j]], ssem.at[b]
            ).wait()
            return carry

        lax.fori_loop(0, 4, drain_body, 0)
        plsc.subcore_barrier()

        pltpu.sync_copy(
            acc.at[pl.ds(s * RPT, RPT)], out_hbm.at[c, pl.ds(s * RPT, RPT)]
        )

    return pl.kernel(
        body,
        out_type=jax.ShapeDtypeStruct((NC, R, width), jnp.float32),
        mesh=_SC_MESH,
        compiler_params=pltpu.CompilerParams(use_tc_tiling_on_sc=False),
        scratch_types=[
            pltpu.VMEM((NCH, CHUNK), jnp.int32),
            pltpu.VMEM((NCH, CHUNK), jnp.int32),
            pltpu.VMEM((4, CHUNK, width), jnp.float32),
            pltpu.VMEM((RPT, width), jnp.float32),
            pltpu.VMEM_SHARED((R, width), jnp.float32),
            pltpu.SemaphoreType.DMA((4,)),
            pltpu.SemaphoreType.DMA((4,)),
        ],
    )


_agg_h = _make_agg(DH)
_agg_o = _make_agg(DO)


_BR = 512
_GRID = R // _BR


def _tc1_body(x_ref, w1_ref, degp_ref, hs_ref):
    deg = jnp.sum(degp_ref[...], axis=0) + 1.0
    dinv = lax.rsqrt(deg)
    h = jnp.dot(x_ref[...], w1_ref[...], preferred_element_type=jnp.float32)
    hs_ref[...] = h * dinv[:, None]


_tc1_call = pl.pallas_call(
    _tc1_body,
    grid=(_GRID,),
    in_specs=[
        pl.BlockSpec((_BR, DF), lambda i: (i, 0)),
        pl.BlockSpec((DF, DH), lambda i: (0, 0)),
        pl.BlockSpec((NW, _BR), lambda i: (0, i)),
    ],
    out_specs=pl.BlockSpec((_BR, DH), lambda i: (i, 0)),
    out_shape=jax.ShapeDtypeStruct((R, DH), jnp.float32),
)


def _tc2_body(acc_ref, hs_ref, degp_ref, wmu_ref, out_ref):
    deg = jnp.sum(degp_ref[...], axis=0) + 1.0
    dinv = lax.rsqrt(deg)
    z = jnp.maximum(
        (acc_ref[0] + acc_ref[1] + hs_ref[...]) * dinv[:, None], 0.0
    )
    h2 = jnp.dot(z, wmu_ref[...], preferred_element_type=jnp.float32)
    out_ref[...] = h2 * dinv[:, None]


_tc2_call = pl.pallas_call(
    _tc2_body,
    grid=(_GRID,),
    in_specs=[
        pl.BlockSpec((NC, _BR, DH), lambda i: (0, i, 0)),
        pl.BlockSpec((_BR, DH), lambda i: (i, 0)),
        pl.BlockSpec((NW, _BR), lambda i: (0, i)),
        pl.BlockSpec((DH, DO), lambda i: (0, 0)),
    ],
    out_specs=pl.BlockSpec((_BR, DO), lambda i: (i, 0)),
    out_shape=jax.ShapeDtypeStruct((R, DO), jnp.float32),
)


def _tc3_body(acc_ref, hs_ref, degp_ref, out_ref):
    deg = jnp.sum(degp_ref[...], axis=0) + 1.0
    dinv = lax.rsqrt(deg)
    out_ref[...] = (acc_ref[0] + acc_ref[1] + hs_ref[...]) * dinv[:, None]


_tc3_call = pl.pallas_call(
    _tc3_body,
    grid=(_GRID,),
    in_specs=[
        pl.BlockSpec((NC, _BR, DO), lambda i: (0, i, 0)),
        pl.BlockSpec((_BR, DO), lambda i: (i, 0)),
        pl.BlockSpec((NW, _BR), lambda i: (0, i)),
    ],
    out_specs=pl.BlockSpec((_BR, DO), lambda i: (i, 0)),
    out_shape=jax.ShapeDtypeStruct((R, DO), jnp.float32),
)


def kernel(x, edge_index, W1, W_mu):
    src = edge_index[0].astype(jnp.int32)
    dst = edge_index[1].astype(jnp.int32)

    npad = EP - E
    pad_idx = N + (jnp.arange(npad, dtype=jnp.int32) % (R - N))
    src_p = jnp.concatenate([src, pad_idx]).reshape(NW, NCH, CHUNK)
    dst_p = jnp.concatenate([dst, pad_idx]).reshape(NW, NCH, CHUNK)
    x_p = jnp.pad(x, ((0, R - N), (0, 0)))

    degp = _deg_call(dst_p)

    hs1 = _tc1_call(x_p, W1, degp)
    acc1 = _agg_h(hs1, src_p, dst_p)

    hs2 = _tc2_call(acc1, hs1, degp, W_mu)
    acc2 = _agg_o(hs2, src_p, dst_p)

    return _tc3_call(acc2, hs2, degp)[:N]
